# baseline (speedup 1.0000x reference)
import functools

import jax
import jax.numpy as jnp
from jax import lax
from jax.experimental import pallas as pl
from jax.experimental.pallas import tpu as pltpu
from jax.experimental.pallas import tpu_sc as plsc

N = 10000
E = 320000
D = 128
H = 64
W = 80
NC = 2
NS = 16
NW = NC * NS
CH = 128
NCHUNK = E // CH
NFULL = NCHUNK // NW
NEXTRA = NCHUNK - NFULL * NW
NPAD = 10240
ROWS_PER_TILE = NPAD // NS

assert NFULL % 2 == 0 and ROWS_PER_TILE % CH == 0


def _node_proj_body(nf_ref, wsT_ref, wrT_ref, ps_ref, pr_ref):
    nf = nf_ref[...]
    ps_ref[...] = jnp.dot(nf, wsT_ref[...], preferred_element_type=jnp.float32,
                          precision=jax.lax.Precision.HIGHEST)
    pr_ref[...] = jnp.dot(nf, wrT_ref[...], preferred_element_type=jnp.float32,
                          precision=jax.lax.Precision.HIGHEST)


def _node_proj(nf, wsT, wrT):
    return pl.pallas_call(
        _node_proj_body,
        out_shape=(
            jax.ShapeDtypeStruct((N, H), jnp.float32),
            jax.ShapeDtypeStruct((N, H), jnp.float32),
        ),
    )(nf, wsT, wrT)


def _edge_proj_body(efc_ref, w2_ref, b2_ref, out_ref):
    efc = efc_ref[...]
    w2 = w2_ref[...]
    b2 = b2_ref[...]
    for t in range(16):
        out_ref[:, t, :] = (
            jnp.dot(efc[:, 8 * t:8 * t + 8], w2,
                    preferred_element_type=jnp.float32,
                    precision=jax.lax.Precision.HIGHEST)
            + b2
        )


def _edge_proj(efc, w2, b2):
    BE = 1000
    grid = (E // 32) // BE
    return pl.pallas_call(
        _edge_proj_body,
        grid=(grid,),
        in_specs=[
            pl.BlockSpec((BE, 128), lambda i: (i, 0)),
            pl.BlockSpec((8, 128), lambda i: (0, 0)),
            pl.BlockSpec((1, 128), lambda i: (0, 0)),
        ],
        out_specs=pl.BlockSpec((BE, 16, 128), lambda i: (i, 0, 0)),
        out_shape=jax.ShapeDtypeStruct((E // 32, 16, 128), jnp.float32),
    )(efc, w2, b2)


def _sc_body(ps_hbm, pr_hbm, ei_hbm, ep_hbm, out_hbm,
             acc,
             sidx_a, ridx_a, hs_a, hr_a, efv_a, msg_a,
             sidx_b, ridx_b, hs_b, hr_b, efv_b, msg_b,
             sem_sa, sem_ra, sem_ea, sem_sb, sem_rb, sem_eb):
    cid = lax.axis_index("c")
    sid = lax.axis_index("s")
    wid = sid * NC + cid

    def zero_row(i, _):
        for k in range(W // 16):
            msg_a[i, pl.ds(k * 16, 16)] = jnp.zeros((16,), jnp.float32)
            msg_b[i, pl.ds(k * 16, 16)] = jnp.zeros((16,), jnp.float32)
        return 0

    lax.fori_loop(0, CH, zero_row, 0)
    for k in range(ROWS_PER_TILE // CH):
        pltpu.sync_copy(msg_a, acc.at[pl.ds(sid * ROWS_PER_TILE + k * CH, CH),
                                      pl.ds(0, W)])
    plsc.subcore_barrier()

    one_lane = jnp.where(lax.iota(jnp.int32, 16) == 0,
                         jnp.float32(1.0), jnp.float32(0.0))

    def one_row(i, _):
        msg_a[i, pl.ds(H, 16)] = one_lane
        msg_b[i, pl.ds(H, 16)] = one_lane
        return 0

    lax.fori_loop(0, CH, one_row, 0)

    def start(t, sidx, ridx, hs, hr, efv, sem_s, sem_r, sem_e):
        off = (t * NW + wid) * CH
        pltpu.sync_copy(ei_hbm.at[0, pl.ds(off, CH)], sidx)
        pltpu.sync_copy(ei_hbm.at[1, pl.ds(off, CH)], ridx)
        pltpu.async_copy(ps_hbm.at[sidx], hs, sem_s)
        pltpu.async_copy(pr_hbm.at[ridx], hr, sem_r)
        pltpu.async_copy(ep_hbm.at[pl.ds(off // 2, CH // 2), :], efv, sem_e)

    def wait(hs, hr, efv, sem_s, sem_r, sem_e):
        pltpu.make_async_copy(ps_hbm.at[sidx_a], hs, sem_s).wait()
        pltpu.make_async_copy(pr_hbm.at[ridx_a], hr, sem_r).wait()
        pltpu.make_async_copy(ep_hbm.at[pl.ds(0, CH // 2), :], efv, sem_e).wait()

    def compute(hs, hr, efv, msg):
        def pair2(q, _):
            for e2 in range(2):
                i = q * 2 + e2
                for k in range(H // 16):
                    sl = pl.ds(k * 16, 16)
                    z = hs[i, sl] + hr[i, sl] + efv[q, pl.ds(e2 * H + k * 16, 16)]
                    msg[i, sl] = jnp.where(z > 0.0, z, jnp.exp(z) - 1.0)
            return 0

        lax.fori_loop(0, CH // 2, pair2, 0)

    def scatter(msg, ridx):
        pltpu.sync_copy(msg, acc.at[ridx], add=True)

    start(0, sidx_a, ridx_a, hs_a, hr_a, efv_a, sem_sa, sem_ra, sem_ea)

    def pair(j, _):
        wait(hs_a, hr_a, efv_a, sem_sa, sem_ra, sem_ea)
        start(2 * j + 1, sidx_b, ridx_b, hs_b, hr_b, efv_b,
              sem_sb, sem_rb, sem_eb)
        compute(hs_a, hr_a, efv_a, msg_a)
        scatter(msg_a, ridx_a)

        wait(hs_b, hr_b, efv_b, sem_sb, sem_rb, sem_eb)

        @pl.when(j < NFULL // 2 - 1)
        def _():
            start(2 * j + 2, sidx_a, ridx_a, hs_a, hr_a, efv_a,
                  sem_sa, sem_ra, sem_ea)

        compute(hs_b, hr_b, efv_b, msg_b)
        scatter(msg_b, ridx_b)
        return 0

    lax.fori_loop(0, NFULL // 2, pair, 0)

    @pl.when(wid < NEXTRA)
    def _():
        start(NFULL, sidx_a, ridx_a, hs_a, hr_a, efv_a, sem_sa, sem_ra, sem_ea)
        wait(hs_a, hr_a, efv_a, sem_sa, sem_ra, sem_ea)
        compute(hs_a, hr_a, efv_a, msg_a)
        scatter(msg_a, ridx_a)

    plsc.subcore_barrier()

    pltpu.sync_copy(
        acc.at[pl.ds(sid * ROWS_PER_TILE, ROWS_PER_TILE), :],
        out_hbm.at[cid, pl.ds(sid * ROWS_PER_TILE, ROWS_PER_TILE), :],
    )


_sc_gather_scatter = functools.partial(
    pl.kernel,
    out_type=jax.ShapeDtypeStruct((NC, NPAD, W), jnp.float32),
    mesh=plsc.VectorSubcoreMesh(core_axis_name="c", subcore_axis_name="s",
                                num_cores=NC, num_subcores=NS),
    scratch_types=[
        pltpu.VMEM_SHARED((NPAD, W), jnp.float32),
        pltpu.VMEM((CH,), jnp.int32),
        pltpu.VMEM((CH,), jnp.int32),
        pltpu.VMEM((CH, H), jnp.float32),
        pltpu.VMEM((CH, H), jnp.float32),
        pltpu.VMEM((CH // 2, 128), jnp.float32),
        pltpu.VMEM((CH, W), jnp.float32),
        pltpu.VMEM((CH,), jnp.int32),
        pltpu.VMEM((CH,), jnp.int32),
        pltpu.VMEM((CH, H), jnp.float32),
        pltpu.VMEM((CH, H), jnp.float32),
        pltpu.VMEM((CH // 2, 128), jnp.float32),
        pltpu.VMEM((CH, W), jnp.float32),
        pltpu.SemaphoreType.DMA,
        pltpu.SemaphoreType.DMA,
        pltpu.SemaphoreType.DMA,
        pltpu.SemaphoreType.DMA,
        pltpu.SemaphoreType.DMA,
        pltpu.SemaphoreType.DMA,
    ],
    compiler_params=pltpu.CompilerParams(use_tc_tiling_on_sc=False),
)(_sc_body)


def _post_body(nf_ref, p0_ref, p1_ref, wm2T_ref, wu1lT_ref, wu1rT_ref,
               bm2_ref, bu1_ref, wu2T_ref, bu2_ref, out_ref):
    p0 = p0_ref[...]
    p1 = p1_ref[...]
    s = p0[:, :H] + p1[:, :H]
    deg = p0[:, H:H + 1] + p1[:, H:H + 1]
    wcT = jnp.dot(wm2T_ref[...], wu1rT_ref[...],
                  preferred_element_type=jnp.float32,
                  precision=jax.lax.Precision.HIGHEST)
    bvec = jnp.dot(bm2_ref[...], wu1rT_ref[...],
                   preferred_element_type=jnp.float32,
                   precision=jax.lax.Precision.HIGHEST)
    u = (jnp.dot(nf_ref[...], wu1lT_ref[...], preferred_element_type=jnp.float32,
                 precision=jax.lax.Precision.HIGHEST)
         + jnp.dot(s, wcT, preferred_element_type=jnp.float32,
                   precision=jax.lax.Precision.HIGHEST)
         + deg * bvec
         + bu1_ref[...])
    h2 = jnp.where(u > 0.0, u, jnp.exp(u) - 1.0)
    out_ref[...] = (jnp.dot(h2, wu2T_ref[...], preferred_element_type=jnp.float32,
                            precision=jax.lax.Precision.HIGHEST)
                    + bu2_ref[...])


def _post(nf, p0, p1, wm2T, wu1lT, wu1rT, bm2, bu1, wu2T, bu2):
    BN = 1000
    grid = N // BN
    wspec = lambda shape: pl.BlockSpec(shape, lambda i: (0, 0))
    return pl.pallas_call(
        _post_body,
        grid=(grid,),
        in_specs=[
            pl.BlockSpec((BN, D), lambda i: (i, 0)),
            pl.BlockSpec((BN, W), lambda i: (i, 0)),
            pl.BlockSpec((BN, W), lambda i: (i, 0)),
            wspec((H, H)),
            wspec((D, H)),
            wspec((H, H)),
            wspec((1, H)),
            wspec((1, H)),
            wspec((H, D)),
            wspec((1, D)),
        ],
        out_specs=pl.BlockSpec((BN, D), lambda i: (i, 0)),
        out_shape=jax.ShapeDtypeStruct((N, D), jnp.float32),
    )(nf, p0, p1, wm2T, wu1lT, wu1rT, bm2.reshape(1, H), bu1.reshape(1, H),
      wu2T, bu2.reshape(1, D))


def kernel(node_features, edge_index, edge_features,
           Wm1, bm1, Wm2, bm2, Wu1, bu1, Wu2, bu2):
    wsT = Wm1[:, :D].T
    wrT = Wm1[:, D:2 * D].T
    weT = Wm1[:, 2 * D:].T

    ps, pr = _node_proj(node_features, wsT, wrT)
    w2 = jnp.kron(jnp.eye(2, dtype=jnp.float32), weT)
    b2 = jnp.concatenate([bm1, bm1]).reshape(1, 128)
    efc = edge_features.reshape(E // 32, 128)
    ep2 = _edge_proj(efc, w2, b2).reshape(E // 2, 128)
    partials = _sc_gather_scatter(ps, pr, edge_index, ep2)

    return _post(
        node_features,
        partials[0],
        partials[1],
        Wm2.T,
        Wu1[:, :D].T,
        Wu1[:, D:].T,
        bm2,
        bu1,
        Wu2.T,
        bu2,
    )


# speedup vs baseline: 1.0042x; 1.0042x over previous
import functools

import jax
import jax.numpy as jnp
from jax import lax
from jax.experimental import pallas as pl
from jax.experimental.pallas import tpu as pltpu
from jax.experimental.pallas import tpu_sc as plsc

N = 10000
E = 320000
D = 128
H = 64
NC = 2
NS = 16
NW = NC * NS
CH = 64
NCHUNK = E // CH
NFULL = NCHUNK // NW
NEXTRA = NCHUNK - NFULL * NW
NPAD = 10112
ROWS_PER_TILE = NPAD // NS

assert NFULL % 2 == 0


def _node_proj_body(nf_ref, wT_ref, p_ref):
    p_ref[...] = jnp.dot(nf_ref[...], wT_ref[...],
                         preferred_element_type=jnp.float32,
                         precision=jax.lax.Precision.HIGHEST)


def _node_proj(nf, wT):
    return pl.pallas_call(
        _node_proj_body,
        out_shape=jax.ShapeDtypeStruct((N, 2 * H), jnp.float32),
    )(nf, wT)


def _edge_proj_body(efc_ref, w2_ref, b2_ref, out_ref):
    efc = efc_ref[...]
    w2 = w2_ref[...]
    b2 = b2_ref[...]
    for t in range(16):
        out_ref[:, t, :] = (
            jnp.dot(efc[:, 8 * t:8 * t + 8], w2,
                    preferred_element_type=jnp.float32,
                    precision=jax.lax.Precision.HIGHEST)
            + b2
        )


def _edge_proj(efc, w2, b2):
    BE = 1000
    grid = (E // 32) // BE
    return pl.pallas_call(
        _edge_proj_body,
        grid=(grid,),
        in_specs=[
            pl.BlockSpec((BE, 128), lambda i: (i, 0)),
            pl.BlockSpec((8, 128), lambda i: (0, 0)),
            pl.BlockSpec((1, 128), lambda i: (0, 0)),
        ],
        out_specs=pl.BlockSpec((BE, 16, 128), lambda i: (i, 0, 0)),
        out_shape=jax.ShapeDtypeStruct((E // 32, 16, 128), jnp.float32),
    )(efc, w2, b2)


def _sc_body(p_hbm, s_hbm, r_hbm, ep_hbm, out_hbm,
             acc,
             sidx_a, ridx_a, hs_a, hr_a, epv_a,
             sidx_b, ridx_b, hs_b, hr_b, epv_b,
             msg,
             sem_sa, sem_ra, sem_ea, sem_sb, sem_rb, sem_eb):
    cid = lax.axis_index("c")
    sid = lax.axis_index("s")
    wid = sid * NC + cid

    def zero_row(i, _):
        for k in range(2 * H // 16):
            msg[i, pl.ds(k * 16, 16)] = jnp.zeros((16,), jnp.float32)
        return 0

    lax.fori_loop(0, CH, zero_row, 0)
    base = sid * ROWS_PER_TILE
    for k in range(ROWS_PER_TILE // CH):
        pltpu.sync_copy(msg, acc.at[pl.ds(base + k * CH, CH), :])
    rem = ROWS_PER_TILE % CH
    if rem:
        pltpu.sync_copy(
            msg.at[pl.ds(0, rem), :],
            acc.at[pl.ds(base + (ROWS_PER_TILE // CH) * CH, rem), :])
    plsc.subcore_barrier()

    one_lane = jnp.where(lax.iota(jnp.int32, 16) == 0,
                         jnp.float32(1.0), jnp.float32(0.0))

    def one_row(i, _):
        msg[i, pl.ds(H, 16)] = one_lane
        return 0

    lax.fori_loop(0, CH, one_row, 0)

    def start(t, sidx, ridx, hs, hr, epv, sem_s, sem_r, sem_e):
        off = (t * NW + wid) * CH
        off2 = pl.multiple_of(off // 2, 8)
        pltpu.sync_copy(s_hbm.at[pl.ds(off, CH)], sidx)
        pltpu.sync_copy(r_hbm.at[pl.ds(off, CH)], ridx)
        pltpu.async_copy(p_hbm.at[sidx], hs, sem_s)
        pltpu.async_copy(p_hbm.at[ridx], hr, sem_r)
        pltpu.async_copy(ep_hbm.at[pl.ds(off2, CH // 2), :], epv, sem_e)

    def wait(hs, hr, epv, sem_s, sem_r, sem_e):
        pltpu.make_async_copy(p_hbm.at[sidx_a], hs, sem_s).wait()
        pltpu.make_async_copy(p_hbm.at[ridx_a], hr, sem_r).wait()
        pltpu.make_async_copy(ep_hbm.at[pl.ds(0, CH // 2), :], epv,
                              sem_e).wait()

    def compute(hs, hr, epv):
        def pair2(q, _):
            for e2 in range(2):
                i = q * 2 + e2
                for k in range(H // 16):
                    z = (hs[i, pl.ds(k * 16, 16)]
                         + hr[i, pl.ds(H + k * 16, 16)]
                         + epv[q, pl.ds(e2 * H + k * 16, 16)])
                    msg[i, pl.ds(k * 16, 16)] = jnp.where(
                        z > 0.0, z, jnp.exp(z) - 1.0)
            return 0

        lax.fori_loop(0, CH // 2, pair2, 0)

    def scatter(ridx):
        pltpu.sync_copy(msg, acc.at[ridx], add=True)

    start(0, sidx_a, ridx_a, hs_a, hr_a, epv_a, sem_sa, sem_ra, sem_ea)

    def pair(j, _):
        wait(hs_a, hr_a, epv_a, sem_sa, sem_ra, sem_ea)
        start(2 * j + 1, sidx_b, ridx_b, hs_b, hr_b, epv_b,
              sem_sb, sem_rb, sem_eb)
        compute(hs_a, hr_a, epv_a)
        scatter(ridx_a)

        wait(hs_b, hr_b, epv_b, sem_sb, sem_rb, sem_eb)

        @pl.when(j < NFULL // 2 - 1)
        def _():
            start(2 * j + 2, sidx_a, ridx_a, hs_a, hr_a, epv_a,
                  sem_sa, sem_ra, sem_ea)

        compute(hs_b, hr_b, epv_b)
        scatter(ridx_b)
        return 0

    lax.fori_loop(0, NFULL // 2, pair, 0)

    @pl.when(wid < NEXTRA)
    def _():
        start(NFULL, sidx_a, ridx_a, hs_a, hr_a, epv_a,
              sem_sa, sem_ra, sem_ea)
        wait(hs_a, hr_a, epv_a, sem_sa, sem_ra, sem_ea)
        compute(hs_a, hr_a, epv_a)
        scatter(ridx_a)

    plsc.subcore_barrier()

    pltpu.sync_copy(
        acc.at[pl.ds(base, ROWS_PER_TILE), :],
        out_hbm.at[cid, pl.ds(base, ROWS_PER_TILE), :],
    )


_sc_gather_scatter = functools.partial(
    pl.kernel,
    out_type=jax.ShapeDtypeStruct((NC, NPAD, 2 * H), jnp.float32),
    mesh=plsc.VectorSubcoreMesh(core_axis_name="c", subcore_axis_name="s",
                                num_cores=NC, num_subcores=NS),
    scratch_types=[
        pltpu.VMEM_SHARED((NPAD, 2 * H), jnp.float32),
        pltpu.VMEM((CH,), jnp.int32),
        pltpu.VMEM((CH,), jnp.int32),
        pltpu.VMEM((CH, 2 * H), jnp.float32),
        pltpu.VMEM((CH, 2 * H), jnp.float32),
        pltpu.VMEM((CH // 2, 128), jnp.float32),
        pltpu.VMEM((CH,), jnp.int32),
        pltpu.VMEM((CH,), jnp.int32),
        pltpu.VMEM((CH, 2 * H), jnp.float32),
        pltpu.VMEM((CH, 2 * H), jnp.float32),
        pltpu.VMEM((CH // 2, 128), jnp.float32),
        pltpu.VMEM((CH, 2 * H), jnp.float32),
        pltpu.SemaphoreType.DMA,
        pltpu.SemaphoreType.DMA,
        pltpu.SemaphoreType.DMA,
        pltpu.SemaphoreType.DMA,
        pltpu.SemaphoreType.DMA,
        pltpu.SemaphoreType.DMA,
    ],
)(_sc_body)


def _post_body(nf_ref, p0_ref, p1_ref, wm2T_ref, wu1lT_ref, wu1rT_ref,
               bm2_ref, bu1_ref, wu2T_ref, bu2_ref, out_ref):
    p0 = p0_ref[...]
    p1 = p1_ref[...]
    s = p0[:, :H] + p1[:, :H]
    deg = p0[:, H:H + 1] + p1[:, H:H + 1]
    wcT = jnp.dot(wm2T_ref[...], wu1rT_ref[...],
                  preferred_element_type=jnp.float32,
                  precision=jax.lax.Precision.HIGHEST)
    bvec = jnp.dot(bm2_ref[...], wu1rT_ref[...],
                   preferred_element_type=jnp.float32,
                   precision=jax.lax.Precision.HIGHEST)
    u = (jnp.dot(nf_ref[...], wu1lT_ref[...],
                 preferred_element_type=jnp.float32,
                 precision=jax.lax.Precision.HIGHEST)
         + jnp.dot(s, wcT, preferred_element_type=jnp.float32,
                   precision=jax.lax.Precision.HIGHEST)
         + deg * bvec
         + bu1_ref[...])
    h2 = jnp.where(u > 0.0, u, jnp.exp(u) - 1.0)
    out_ref[...] = (jnp.dot(h2, wu2T_ref[...],
                            preferred_element_type=jnp.float32,
                            precision=jax.lax.Precision.HIGHEST)
                    + bu2_ref[...])


def _post(nf, p0, p1, wm2T, wu1lT, wu1rT, bm2, bu1, wu2T, bu2):
    BN = 1000
    grid = N // BN
    wspec = lambda shape: pl.BlockSpec(shape, lambda i: (0, 0))
    return pl.pallas_call(
        _post_body,
        grid=(grid,),
        in_specs=[
            pl.BlockSpec((BN, D), lambda i: (i, 0)),
            pl.BlockSpec((BN, 2 * H), lambda i: (i, 0)),
            pl.BlockSpec((BN, 2 * H), lambda i: (i, 0)),
            wspec((H, H)),
            wspec((D, H)),
            wspec((H, H)),
            wspec((1, H)),
            wspec((1, H)),
            wspec((H, D)),
            wspec((1, D)),
        ],
        out_specs=pl.BlockSpec((BN, D), lambda i: (i, 0)),
        out_shape=jax.ShapeDtypeStruct((N, D), jnp.float32),
    )(nf, p0, p1, wm2T, wu1lT, wu1rT, bm2.reshape(1, H), bu1.reshape(1, H),
      wu2T, bu2.reshape(1, D))


def kernel(node_features, edge_index, edge_features,
           Wm1, bm1, Wm2, bm2, Wu1, bu1, Wu2, bu2):
    weT = Wm1[:, 2 * D:].T
    wT = jnp.concatenate([Wm1[:, :D].T, Wm1[:, D:2 * D].T], axis=1)

    p = _node_proj(node_features, wT)
    w2 = jnp.kron(jnp.eye(2, dtype=jnp.float32), weT)
    b2 = jnp.concatenate([bm1, bm1]).reshape(1, 128)
    efc = edge_features.reshape(E // 32, 128)
    ep2 = _edge_proj(efc, w2, b2).reshape(E // 2, 128)
    partials = _sc_gather_scatter(p, edge_index[0], edge_index[1], ep2)

    return _post(
        node_features,
        partials[0],
        partials[1],
        Wm2.T,
        Wu1[:, :D].T,
        Wu1[:, D:].T,
        bm2,
        bu1,
        Wu2.T,
        bu2,
    )


# speedup vs baseline: 1.3955x; 1.3898x over previous
import functools

import jax
import jax.numpy as jnp
from jax import lax
from jax.experimental import pallas as pl
from jax.experimental.pallas import tpu as pltpu
from jax.experimental.pallas import tpu_sc as plsc

N = 10000
E = 320000
D = 128
H = 64
NC = 2
NS = 16
NW = NC * NS
CH = 64
NCHUNK = E // CH
NFULL = NCHUNK // NW
NEXTRA = NCHUNK - NFULL * NW
NPAD = 10240
ROWS_PER_TILE = NPAD // NS


def _node_proj_body(nf_ref, wT_ref, p_ref):
    p_ref[...] = jnp.dot(nf_ref[...], wT_ref[...],
                         preferred_element_type=jnp.float32,
                         precision=jax.lax.Precision.HIGHEST)


def _node_proj(nf, wT):
    return pl.pallas_call(
        _node_proj_body,
        out_shape=jax.ShapeDtypeStruct((N, 2 * H), jnp.float32),
    )(nf, wT)


def _edge_proj_body(ef_ref, weT_ref, b_ref, out_ref):
    out_ref[...] = (
        jnp.dot(ef_ref[...], weT_ref[...], preferred_element_type=jnp.float32,
                precision=jax.lax.Precision.HIGHEST)
        + b_ref[...]
    )


def _edge_proj(ef, weT, bm1):
    BE = 4000
    grid = E // BE
    return pl.pallas_call(
        _edge_proj_body,
        grid=(grid,),
        in_specs=[
            pl.BlockSpec((BE, 4), lambda i: (i, 0)),
            pl.BlockSpec((4, H), lambda i: (0, 0)),
            pl.BlockSpec((1, H), lambda i: (0, 0)),
        ],
        out_specs=pl.BlockSpec((BE, H), lambda i: (i, 0)),
        out_shape=jax.ShapeDtypeStruct((E, H), jnp.float32),
    )(ef, weT, bm1.reshape(1, H))


def _sc_body(p_hbm, ep_hbm, s_hbm, r_hbm, out_hbm,
             acc, sidx, ridx, hs, hr, ep, msg, sem_s, sem_r, sem_e):
    cid = lax.axis_index("c")
    sid = lax.axis_index("s")
    wid = sid * NC + cid

    def zero_row(i, _):
        for k in range(2 * H // 16):
            msg[i, pl.ds(k * 16, 16)] = jnp.zeros((16,), jnp.float32)
        return 0

    lax.fori_loop(0, CH, zero_row, 0)
    for k in range(ROWS_PER_TILE // CH):
        pltpu.sync_copy(msg, acc.at[pl.ds(sid * ROWS_PER_TILE + k * CH, CH), :])
    plsc.subcore_barrier()

    one_lane = jnp.where(lax.iota(jnp.int32, 16) == 0,
                         jnp.float32(1.0), jnp.float32(0.0))

    def one_row(i, _):
        msg[i, pl.ds(H, 16)] = one_lane
        return 0

    lax.fori_loop(0, CH, one_row, 0)

    nchunks = NFULL + jnp.where(wid < NEXTRA, 1, 0)

    def chunk_body(it, _):
        off = (it * NW + wid) * CH
        pltpu.sync_copy(s_hbm.at[pl.ds(off, CH)], sidx)
        pltpu.sync_copy(r_hbm.at[pl.ds(off, CH)], ridx)
        cp_s = pltpu.async_copy(p_hbm.at[sidx], hs, sem_s)
        cp_r = pltpu.async_copy(p_hbm.at[ridx], hr, sem_r)
        cp_e = pltpu.async_copy(ep_hbm.at[pl.ds(off, CH), :], ep, sem_e)
        cp_s.wait()
        cp_r.wait()
        cp_e.wait()

        def row(i, _):
            for k in range(H // 16):
                sl = pl.ds(k * 16, 16)
                z = hs[i, sl] + hr[i, pl.ds(H + k * 16, 16)] + ep[i, sl]
                msg[i, sl] = jnp.where(z > 0.0, z, jnp.exp(z) - 1.0)
            return 0

        lax.fori_loop(0, CH, row, 0)
        pltpu.sync_copy(msg, acc.at[ridx], add=True)
        return 0

    lax.fori_loop(0, nchunks, chunk_body, 0)
    plsc.subcore_barrier()

    pltpu.sync_copy(
        acc.at[pl.ds(sid * ROWS_PER_TILE, ROWS_PER_TILE), :],
        out_hbm.at[cid, pl.ds(sid * ROWS_PER_TILE, ROWS_PER_TILE), :],
    )


_sc_gather_scatter = functools.partial(
    pl.kernel,
    out_type=jax.ShapeDtypeStruct((NC, NPAD, 2 * H), jnp.float32),
    mesh=plsc.VectorSubcoreMesh(core_axis_name="c", subcore_axis_name="s",
                                num_cores=NC, num_subcores=NS),
    scratch_types=[
        pltpu.VMEM_SHARED((NPAD, 2 * H), jnp.float32),
        pltpu.VMEM((CH,), jnp.int32),
        pltpu.VMEM((CH,), jnp.int32),
        pltpu.VMEM((CH, 2 * H), jnp.float32),
        pltpu.VMEM((CH, 2 * H), jnp.float32),
        pltpu.VMEM((CH, H), jnp.float32),
        pltpu.VMEM((CH, 2 * H), jnp.float32),
        pltpu.SemaphoreType.DMA,
        pltpu.SemaphoreType.DMA,
        pltpu.SemaphoreType.DMA,
    ],
)(_sc_body)


def _post_body(nf_ref, p0_ref, p1_ref, wm2T_ref, wu1lT_ref, wu1rT_ref,
               bm2_ref, bu1_ref, wu2T_ref, bu2_ref, out_ref):
    p0 = p0_ref[...]
    p1 = p1_ref[...]
    s = p0[:, :H] + p1[:, :H]
    deg = p0[:, H:H + 1] + p1[:, H:H + 1]
    wcT = jnp.dot(wm2T_ref[...], wu1rT_ref[...],
                  preferred_element_type=jnp.float32,
                  precision=jax.lax.Precision.HIGHEST)
    bvec = jnp.dot(bm2_ref[...], wu1rT_ref[...],
                   preferred_element_type=jnp.float32,
                   precision=jax.lax.Precision.HIGHEST)
    u = (jnp.dot(nf_ref[...], wu1lT_ref[...],
                 preferred_element_type=jnp.float32,
                 precision=jax.lax.Precision.HIGHEST)
         + jnp.dot(s, wcT, preferred_element_type=jnp.float32,
                   precision=jax.lax.Precision.HIGHEST)
         + deg * bvec
         + bu1_ref[...])
    h2 = jnp.where(u > 0.0, u, jnp.exp(u) - 1.0)
    out_ref[...] = (jnp.dot(h2, wu2T_ref[...],
                            preferred_element_type=jnp.float32,
                            precision=jax.lax.Precision.HIGHEST)
                    + bu2_ref[...])


def _post(nf, p0, p1, wm2T, wu1lT, wu1rT, bm2, bu1, wu2T, bu2):
    BN = 1000
    grid = N // BN
    wspec = lambda shape: pl.BlockSpec(shape, lambda i: (0, 0))
    return pl.pallas_call(
        _post_body,
        grid=(grid,),
        in_specs=[
            pl.BlockSpec((BN, D), lambda i: (i, 0)),
            pl.BlockSpec((BN, 2 * H), lambda i: (i, 0)),
            pl.BlockSpec((BN, 2 * H), lambda i: (i, 0)),
            wspec((H, H)),
            wspec((D, H)),
            wspec((H, H)),
            wspec((1, H)),
            wspec((1, H)),
            wspec((H, D)),
            wspec((1, D)),
        ],
        out_specs=pl.BlockSpec((BN, D), lambda i: (i, 0)),
        out_shape=jax.ShapeDtypeStruct((N, D), jnp.float32),
    )(nf, p0, p1, wm2T, wu1lT, wu1rT, bm2.reshape(1, H), bu1.reshape(1, H),
      wu2T, bu2.reshape(1, D))


def kernel(node_features, edge_index, edge_features,
           Wm1, bm1, Wm2, bm2, Wu1, bu1, Wu2, bu2):
    senders = edge_index[0]
    receivers = edge_index[1]

    wT = jnp.concatenate([Wm1[:, :D].T, Wm1[:, D:2 * D].T], axis=1)
    weT = Wm1[:, 2 * D:].T

    p = _node_proj(node_features, wT)
    ep = _edge_proj(edge_features, weT, bm1)
    partials = _sc_gather_scatter(p, ep, senders, receivers)

    return _post(
        node_features,
        partials[0],
        partials[1],
        Wm2.T,
        Wu1[:, :D].T,
        Wu1[:, D:].T,
        bm2,
        bu1,
        Wu2.T,
        bu2,
    )
